# Initial kernel scaffold; baseline (speedup 1.0000x reference)
import jax
import jax.numpy as jnp
from jax.experimental import pallas as pl


def kernel(x, W_E):
    raise NotImplementedError("write your pallas kernel here")


# speedup vs baseline: 8.1687x; 8.1687x over previous
import functools

import jax
import jax.numpy as jnp
from jax import lax
from jax.experimental import pallas as pl
from jax.experimental.pallas import tpu as pltpu
from jax.experimental.pallas import tpu_sc as plsc

D_EMBED = 128
NUM_CORES = 2
NUM_SUBCORES = 16
NUM_WORKERS = NUM_CORES * NUM_SUBCORES
CHUNK = 512


def _embed_body(n_chunks, table_hbm, idx_hbm, out_hbm, idx_v, rows_v, sem):
    wid = lax.axis_index("s") * NUM_CORES + lax.axis_index("c")
    base = wid * (n_chunks * CHUNK)

    def body(i, carry):
        off = base + i * CHUNK
        pltpu.sync_copy(idx_hbm.at[pl.ds(off, CHUNK)], idx_v)
        pltpu.async_copy(table_hbm.at[idx_v], rows_v, sem).wait()
        pltpu.sync_copy(rows_v, out_hbm.at[pl.ds(off, CHUNK)])
        return carry

    lax.fori_loop(0, n_chunks, body, 0)


@functools.partial(jax.jit, static_argnums=(2,))
def _embed(x_flat, w, b_total):
    n_chunks = b_total // (NUM_WORKERS * CHUNK)
    mesh = plsc.VectorSubcoreMesh(core_axis_name="c", subcore_axis_name="s")
    run = pl.kernel(
        functools.partial(_embed_body, n_chunks),
        out_type=jax.ShapeDtypeStruct((b_total, D_EMBED), jnp.float32),
        mesh=mesh,
        scratch_types=[
            pltpu.VMEM((CHUNK,), jnp.int32),
            pltpu.VMEM((CHUNK, D_EMBED), jnp.float32),
            pltpu.SemaphoreType.DMA,
        ],
    )
    return run(w, x_flat)


def kernel(x, W_E):
    batch, seq = x.shape
    x_flat = x.reshape(-1).astype(jnp.int32)
    out = _embed(x_flat, W_E, batch * seq)
    return out.reshape(batch, seq, D_EMBED)


# speedup vs baseline: 8.9342x; 1.0937x over previous
import functools

import jax
import jax.numpy as jnp
from jax import lax
from jax.experimental import pallas as pl
from jax.experimental.pallas import tpu as pltpu
from jax.experimental.pallas import tpu_sc as plsc

D_EMBED = 128
NUM_CORES = 2
NUM_SUBCORES = 16
NUM_WORKERS = NUM_CORES * NUM_SUBCORES
CHUNK = 400


def _embed_body(n_chunks, table, idx_hbm, out_hbm,
                idx_v, rows0, rows1, sg0, sg1, ss0, ss1):
    wid = lax.axis_index("s") * NUM_CORES + lax.axis_index("c")
    base = wid * (n_chunks * CHUNK)
    pltpu.sync_copy(idx_hbm.at[pl.ds(base, n_chunks * CHUNK)], idx_v)

    def gather(c, buf, sem):
        return pltpu.make_async_copy(
            table.at[idx_v.at[pl.ds(c * CHUNK, CHUNK)]], buf, sem)

    def store(c, buf, sem):
        return pltpu.make_async_copy(
            buf, out_hbm.at[pl.ds(base + c * CHUNK, CHUNK)], sem)

    gather(0, rows0, sg0).start()
    gather(1, rows1, sg1).start()

    n_pairs = n_chunks // 2

    def body(j, carry):
        g = j * 2
        gather(g, rows0, sg0).wait()
        store(g, rows0, ss0).start()
        gather(g + 1, rows1, sg1).wait()
        store(g + 1, rows1, ss1).start()

        @pl.when(j + 1 < n_pairs)
        def _():
            store(g, rows0, ss0).wait()
            gather(g + 2, rows0, sg0).start()
            store(g + 1, rows1, ss1).wait()
            gather(g + 3, rows1, sg1).start()

        return carry

    lax.fori_loop(0, n_pairs, body, 0)
    store(n_chunks - 2, rows0, ss0).wait()
    store(n_chunks - 1, rows1, ss1).wait()


@functools.partial(jax.jit, static_argnums=(2,))
def _embed(x_flat, w, b_total):
    n_chunks = b_total // (NUM_WORKERS * CHUNK)
    mesh = plsc.VectorSubcoreMesh(core_axis_name="c", subcore_axis_name="s")
    run = pl.kernel(
        functools.partial(_embed_body, n_chunks),
        out_type=jax.ShapeDtypeStruct((b_total, D_EMBED), jnp.float32),
        mesh=mesh,
        scratch_types=[
            pltpu.VMEM((n_chunks * CHUNK,), jnp.int32),
            pltpu.VMEM((CHUNK, D_EMBED), jnp.float32),
            pltpu.VMEM((CHUNK, D_EMBED), jnp.float32),
            pltpu.SemaphoreType.DMA,
            pltpu.SemaphoreType.DMA,
            pltpu.SemaphoreType.DMA,
            pltpu.SemaphoreType.DMA,
        ],
    )
    return run(w, x_flat)


def kernel(x, W_E):
    batch, seq = x.shape
    x_flat = x.reshape(-1).astype(jnp.int32)
    out = _embed(x_flat, W_E, batch * seq)
    return out.reshape(batch, seq, D_EMBED)


# speedup vs baseline: 9.2279x; 1.0329x over previous
import functools

import jax
import jax.numpy as jnp
from jax import lax
from jax.experimental import pallas as pl
from jax.experimental.pallas import tpu as pltpu
from jax.experimental.pallas import tpu_sc as plsc

D_EMBED = 128
NUM_CORES = 2
NUM_SUBCORES = 16
NUM_WORKERS = NUM_CORES * NUM_SUBCORES
CHUNK = 200
NBUF = 4


def _embed_body(n_chunks, table, idx_hbm, out_hbm,
                idx_v, b0, b1, b2, b3,
                sg0, sg1, sg2, sg3, ss0, ss1, ss2, ss3):
    bufs = (b0, b1, b2, b3)
    sg = (sg0, sg1, sg2, sg3)
    ss = (ss0, ss1, ss2, ss3)

    wid = lax.axis_index("s") * NUM_CORES + lax.axis_index("c")
    base = wid * (n_chunks * CHUNK)
    pltpu.sync_copy(idx_hbm.at[pl.ds(base, n_chunks * CHUNK)], idx_v)

    def gather(c, buf, sem):
        return pltpu.make_async_copy(
            table.at[idx_v.at[pl.ds(c * CHUNK, CHUNK)]], buf, sem)

    def store(c, buf, sem):
        return pltpu.make_async_copy(
            buf, out_hbm.at[pl.ds(base + c * CHUNK, CHUNK)], sem)

    gather(0, bufs[0], sg[0]).start()
    gather(1, bufs[1], sg[1]).start()
    gather(2, bufs[2], sg[2]).start()

    def body(k, carry):
        for b in range(NBUF):
            c = k * NBUF + b
            gather(c, bufs[b], sg[b]).wait()
            store(c, bufs[b], ss[b]).start()
            g = c + NBUF - 1
            bg = (b + NBUF - 1) % NBUF

            @pl.when(jnp.logical_and(g < n_chunks, c > 0))
            def _():
                store(c - 1, bufs[bg], ss[bg]).wait()

            @pl.when(g < n_chunks)
            def _():
                gather(g, bufs[bg], sg[bg]).start()

        return carry

    lax.fori_loop(0, n_chunks // NBUF, body, 0)

    for i in range(NBUF):
        c = n_chunks - NBUF + i
        store(c, bufs[c % NBUF], ss[c % NBUF]).wait()


@functools.partial(jax.jit, static_argnums=(2,))
def _embed(x_flat, w, b_total):
    n_chunks = b_total // (NUM_WORKERS * CHUNK)
    mesh = plsc.VectorSubcoreMesh(core_axis_name="c", subcore_axis_name="s")
    run = pl.kernel(
        functools.partial(_embed_body, n_chunks),
        out_type=jax.ShapeDtypeStruct((b_total, D_EMBED), jnp.float32),
        mesh=mesh,
        scratch_types=(
            [pltpu.VMEM((n_chunks * CHUNK,), jnp.int32)]
            + [pltpu.VMEM((CHUNK, D_EMBED), jnp.float32) for _ in range(NBUF)]
            + [pltpu.SemaphoreType.DMA for _ in range(2 * NBUF)]
        ),
    )
    return run(w, x_flat)


def kernel(x, W_E):
    batch, seq = x.shape
    x_flat = x.reshape(-1).astype(jnp.int32)
    out = _embed(x_flat, W_E, batch * seq)
    return out.reshape(batch, seq, D_EMBED)


# speedup vs baseline: 9.2317x; 1.0004x over previous
import functools

import jax
import jax.numpy as jnp
from jax import lax
from jax.experimental import pallas as pl
from jax.experimental.pallas import tpu as pltpu
from jax.experimental.pallas import tpu_sc as plsc

D_EMBED = 128
NUM_CORES = 2
NUM_SUBCORES = 16
NUM_WORKERS = NUM_CORES * NUM_SUBCORES
CHUNK = 160
NBUF = 5


def _embed_body(n_chunks, table, idx_hbm, out_hbm, *refs):
    bufs = refs[1:1 + NBUF]
    sg = refs[1 + NBUF:1 + 2 * NBUF]
    ss = refs[1 + 2 * NBUF:1 + 3 * NBUF]
    idx_v = refs[0]

    wid = lax.axis_index("s") * NUM_CORES + lax.axis_index("c")
    base = wid * (n_chunks * CHUNK)
    pltpu.sync_copy(idx_hbm.at[pl.ds(base, n_chunks * CHUNK)], idx_v)

    def gather(c, buf, sem):
        return pltpu.make_async_copy(
            table.at[idx_v.at[pl.ds(c * CHUNK, CHUNK)]], buf, sem)

    def store(c, buf, sem):
        return pltpu.make_async_copy(
            buf, out_hbm.at[pl.ds(base + c * CHUNK, CHUNK)], sem)

    for b in range(NBUF - 1):
        gather(b, bufs[b], sg[b]).start()

    def body(k, carry):
        for b in range(NBUF):
            c = k * NBUF + b
            gather(c, bufs[b], sg[b]).wait()
            store(c, bufs[b], ss[b]).start()
            g = c + NBUF - 1
            bg = (b + NBUF - 1) % NBUF

            @pl.when(jnp.logical_and(g < n_chunks, c > 0))
            def _():
                store(c - 1, bufs[bg], ss[bg]).wait()

            @pl.when(g < n_chunks)
            def _():
                gather(g, bufs[bg], sg[bg]).start()

        return carry

    lax.fori_loop(0, n_chunks // NBUF, body, 0)

    for i in range(NBUF):
        c = n_chunks - NBUF + i
        store(c, bufs[c % NBUF], ss[c % NBUF]).wait()


@functools.partial(jax.jit, static_argnums=(2,))
def _embed(x_flat, w, b_total):
    n_chunks = b_total // (NUM_WORKERS * CHUNK)
    mesh = plsc.VectorSubcoreMesh(core_axis_name="c", subcore_axis_name="s")
    run = pl.kernel(
        functools.partial(_embed_body, n_chunks),
        out_type=jax.ShapeDtypeStruct((b_total, D_EMBED), jnp.float32),
        mesh=mesh,
        scratch_types=(
            [pltpu.VMEM((n_chunks * CHUNK,), jnp.int32)]
            + [pltpu.VMEM((CHUNK, D_EMBED), jnp.float32) for _ in range(NBUF)]
            + [pltpu.SemaphoreType.DMA for _ in range(2 * NBUF)]
        ),
    )
    return run(w, x_flat)


def kernel(x, W_E):
    batch, seq = x.shape
    x_flat = x.reshape(-1).astype(jnp.int32)
    out = _embed(x_flat, W_E, batch * seq)
    return out.reshape(batch, seq, D_EMBED)
